# Initial kernel scaffold; baseline (speedup 1.0000x reference)
import jax
import jax.numpy as jnp
from jax.experimental import pallas as pl


def kernel(x, edge_index, W1, b1, W2, b2, fc1_W, fc1_b, fc2_W, fc2_b):
    raise NotImplementedError("write your pallas kernel here")


# speedup vs baseline: 7.6098x; 7.6098x over previous
import functools

import jax
import jax.numpy as jnp
from jax import lax
from jax.experimental import pallas as pl
from jax.experimental.pallas import tpu as pltpu
from jax.experimental.pallas import tpu_sc as plsc

N = 50000
E = 800000
NP = 50176
EP = 819200
CH = 2048
NCHUNK = 25
DCH = 1280
NDCH = 20
D1 = 256
D2 = 896

_mesh = plsc.VectorSubcoreMesh(core_axis_name="c", subcore_axis_name="s")
_sc_params = pltpu.CompilerParams(needs_layout_passes=False)


def _sc_degree(dst_p, zdeg, ones16):
    FST = NP // 8

    @functools.partial(
        pl.kernel,
        out_type=jax.ShapeDtypeStruct((2 * NP,), jnp.float32),
        mesh=_mesh,
        compiler_params=_sc_params,
        scratch_types=[
            pltpu.VMEM((DCH,), jnp.int32),
            pltpu.VMEM((128,), jnp.float32),
            pltpu.VMEM((NP // 8,), jnp.float32),
            pltpu.VMEM_SHARED((NP,), jnp.float32),
        ],
    )
    def deg_kernel(dst_hbm, zdeg_hbm, ones_hbm, out_hbm, dbuf, ones_v,
                   hbounce, hist):
        core = lax.axis_index("c")
        sub = lax.axis_index("s")
        wid = sub * 2 + core

        @pl.when(sub < 8)
        def _():
            pltpu.sync_copy(zdeg_hbm, hbounce)
            pltpu.sync_copy(hbounce, hist.at[pl.ds(sub * FST, FST)])

        pltpu.sync_copy(ones_hbm, ones_v)
        plsc.subcore_barrier()
        base = wid * (EP // 32)

        def chunk(c, carry):
            pltpu.sync_copy(dst_hbm.at[pl.ds(base + c * DCH, DCH)], dbuf)

            def grp(g, carry2):
                idx = dbuf.at[pl.ds(g * 128, 128)]
                pltpu.sync_copy(ones_v, hist.at[idx], add=True)
                return carry2

            return lax.fori_loop(0, DCH // 128, grp, carry)

        lax.fori_loop(0, NDCH, chunk, jnp.int32(0))
        plsc.subcore_barrier()

        @pl.when(sub < 8)
        def _():
            pltpu.sync_copy(hist.at[pl.ds(sub * FST, FST)], hbounce)
            pltpu.sync_copy(
                hbounce, out_hbm.at[pl.ds(core * NP + sub * FST, FST)])

    return deg_kernel(dst_p, zdeg, ones16)


def _make_sc_segsum(D, FT, ZB, NZ):
    OWN = NP // 32

    @functools.partial(
        pl.kernel,
        out_type=jax.ShapeDtypeStruct((NP + 32, D), jnp.float32),
        mesh=_mesh,
        compiler_params=_sc_params,
        scratch_types=[
            pltpu.VMEM((CH,), jnp.int32),
            pltpu.VMEM((CH,), jnp.int32),
            pltpu.VMEM((CH + FT,), jnp.int32),
            pltpu.VMEM((CH + FT,), jnp.int32),
            pltpu.VMEM((FT, D), jnp.float32),
            pltpu.VMEM((ZB, D), jnp.float32),
        ],
    )
    def seg_kernel(u_hbm, src_hbm, dst_hbm, zblk_hbm, out_hbm,
                   ebuf_s, ebuf_d, slist, dlist, stage, zbuf):
        core = lax.axis_index("c")
        sub = lax.axis_index("s")
        wid = sub * 2 + core
        lo = wid * OWN
        garbage = NP + wid
        pltpu.sync_copy(zblk_hbm, zbuf)
        for z in range(NZ):
            pltpu.sync_copy(zbuf, out_hbm.at[pl.ds(lo + z * ZB, ZB)])

        def per_chunk(c, ptr_in):
            pltpu.sync_copy(src_hbm.at[pl.ds(c * CH, CH)], ebuf_s)
            pltpu.sync_copy(dst_hbm.at[pl.ds(c * CH, CH)], ebuf_d)

            def per_group(g, ptr):
                dv = ebuf_d[pl.ds(g * 16, 16)]
                sv = ebuf_s[pl.ds(g * 16, 16)]
                m = (dv >= lo) & (dv < lo + OWN)
                mi = m.astype(jnp.int32)
                pos = ptr + plsc.cumsum(mi) - 1
                plsc.store_scatter(slist, [pos], sv, mask=m)
                plsc.store_scatter(dlist, [pos], dv, mask=m)
                return ptr + jnp.sum(mi)

            ptr = lax.fori_loop(0, CH // 16, per_group, ptr_in)
            k = ptr // FT

            def flush(j, carry):
                sref = slist.at[pl.ds(j * FT, FT)]
                dref = dlist.at[pl.ds(j * FT, FT)]
                pltpu.sync_copy(u_hbm.at[sref], stage)
                pltpu.sync_copy(stage, out_hbm.at[dref], add=True)
                return carry

            lax.fori_loop(0, k, flush, jnp.int32(0))
            for t in range(FT // 16):
                slist[pl.ds(t * 16, 16)] = slist[pl.ds(k * FT + t * 16, 16)]
                dlist[pl.ds(t * 16, 16)] = dlist[pl.ds(k * FT + t * 16, 16)]
            return ptr - k * FT

        ptrf = lax.fori_loop(0, EP // CH, per_chunk, jnp.int32(0))
        lane = lax.iota(jnp.int32, 16)
        for t in range(FT // 16):
            mt = (lane + t * 16) < ptrf
            slist[pl.ds(t * 16, 16)] = jnp.where(
                mt, slist[pl.ds(t * 16, 16)], 0)
            dlist[pl.ds(t * 16, 16)] = jnp.where(
                mt, dlist[pl.ds(t * 16, 16)], garbage)
        pltpu.sync_copy(u_hbm.at[slist.at[pl.ds(0, FT)]], stage)
        pltpu.sync_copy(stage, out_hbm.at[dlist.at[pl.ds(0, FT)]], add=True)

    return seg_kernel


_sc_segsum_1 = _make_sc_segsum(D1, 64, 112, 14)
_sc_segsum_2 = _make_sc_segsum(D2, 64, 32, 49)


def _tc_prep_body(dega_ref, degb_ref, x_ref, dinv_ref, u0_ref):
    deg = dega_ref[...] + degb_ref[...] + 1.0
    dv = lax.rsqrt(deg)
    dinv_ref[...] = dv
    u0_ref[...] = x_ref[...] * dv


def _tc_prep(H, xp):
    R, G = 3136, 16
    degc = H.reshape(2 * NP, 1)
    return pl.pallas_call(
        _tc_prep_body,
        grid=(G,),
        in_specs=[
            pl.BlockSpec((R, 1), lambda i: (i, 0)),
            pl.BlockSpec((R, 1), lambda i: (i + 16, 0)),
            pl.BlockSpec((R, D1), lambda i: (i, 0)),
        ],
        out_specs=[
            pl.BlockSpec((R, 1), lambda i: (i, 0)),
            pl.BlockSpec((R, D1), lambda i: (i, 0)),
        ],
        out_shape=[
            jax.ShapeDtypeStruct((NP, 1), jnp.float32),
            jax.ShapeDtypeStruct((NP, D1), jnp.float32),
        ],
    )(degc, degc, xp)


def _tc_layer1_body(s0_ref, u0_ref, dinv_ref, w1_ref, b1_ref, u1_ref):
    agg = dinv_ref[...] * (s0_ref[...] + u0_ref[...])
    z = jnp.dot(agg, w1_ref[...], preferred_element_type=jnp.float32,
                precision=lax.Precision.HIGHEST)
    h = jnp.maximum(z + b1_ref[...], 0.0)
    u1_ref[...] = dinv_ref[...] * h


def _tc_layer1(S0, u0, dinv, W1p, b1p):
    R, G = 1568, 32
    return pl.pallas_call(
        _tc_layer1_body,
        grid=(G,),
        in_specs=[
            pl.BlockSpec((R, D1), lambda i: (i, 0)),
            pl.BlockSpec((R, D1), lambda i: (i, 0)),
            pl.BlockSpec((R, 1), lambda i: (i, 0)),
            pl.BlockSpec((D1, D2), lambda i: (0, 0)),
            pl.BlockSpec((1, D2), lambda i: (0, 0)),
        ],
        out_specs=pl.BlockSpec((R, D2), lambda i: (i, 0)),
        out_shape=jax.ShapeDtypeStruct((NP, D2), jnp.float32),
    )(S0, u0, dinv, W1p, b1p)


def _make_tc_final_body(R, G):
    def body(s1_ref, u1_ref, dinv_ref, w2_ref, b2_ref,
             f1w_ref, f1b_ref, f2w_ref, f2b_ref, out_ref, acc_ref):
        i = pl.program_id(0)

        @pl.when(i == 0)
        def _():
            acc_ref[...] = jnp.zeros_like(acc_ref)

        agg = dinv_ref[...] * (s1_ref[...] + u1_ref[...])
        z = jnp.dot(agg, w2_ref[...], preferred_element_type=jnp.float32,
                    precision=lax.Precision.HIGHEST)
        h = jnp.maximum(z + b2_ref[...], 0.0)
        rid = i * R + lax.broadcasted_iota(jnp.int32, (R, 1), 0)
        h = jnp.where(rid < N, h, 0.0)
        acc_ref[...] += jnp.sum(h, axis=0, keepdims=True)

        @pl.when(i == G - 1)
        def _():
            m = acc_ref[...] / float(N)
            t = jnp.dot(m, f1w_ref[...], preferred_element_type=jnp.float32,
                        precision=lax.Precision.HIGHEST)
            t = t + f1b_ref[...]
            t = jnp.where(t > 0, t, 0.2 * t)
            o = jnp.dot(t, f2w_ref[...], preferred_element_type=jnp.float32,
                        precision=lax.Precision.HIGHEST)
            o = o + f2b_ref[...]
            out_ref[...] = 1.0 / (1.0 + jnp.exp(-o))

    return body


def _tc_final(S1, u1, dinv, W2p, b2, fc1_W, fc1_b, fc2_W, fc2_b):
    R, G = 1568, 32
    return pl.pallas_call(
        _make_tc_final_body(R, G),
        grid=(G,),
        in_specs=[
            pl.BlockSpec((R, D2), lambda i: (i, 0)),
            pl.BlockSpec((R, D2), lambda i: (i, 0)),
            pl.BlockSpec((R, 1), lambda i: (i, 0)),
            pl.BlockSpec((D2, 1024), lambda i: (0, 0)),
            pl.BlockSpec((1, 1024), lambda i: (0, 0)),
            pl.BlockSpec((1024, 512), lambda i: (0, 0)),
            pl.BlockSpec((1, 512), lambda i: (0, 0)),
            pl.BlockSpec((512, 1), lambda i: (0, 0)),
            pl.BlockSpec((1, 1), lambda i: (0, 0)),
        ],
        out_specs=pl.BlockSpec((1, 1), lambda i: (0, 0)),
        out_shape=jax.ShapeDtypeStruct((1, 1), jnp.float32),
        scratch_shapes=[pltpu.VMEM((1, 1024), jnp.float32)],
    )(S1, u1, dinv, W2p, b2.reshape(1, 1024), fc1_W,
      fc1_b.reshape(1, 512), fc2_W, fc2_b.reshape(1, 1))


def kernel(x, edge_index, W1, b1, W2, b2, fc1_W, fc1_b, fc2_W, fc2_b):
    xp = jnp.pad(x, ((0, NP - N), (0, D1 - 78)))
    src_p = jnp.pad(edge_index[0], (0, EP - E), constant_values=0)
    dst_p = jnp.pad(edge_index[1], (0, EP - E), constant_values=NP)
    W1p = jnp.pad(W1, ((0, D1 - 78), (0, D2 - 780)))
    b1p = jnp.pad(b1, (0, D2 - 780)).reshape(1, D2)
    W2p = jnp.pad(W2, ((0, D2 - 780), (0, 0)))

    zdeg = jnp.zeros((NP // 8,), jnp.float32)
    ones16 = jnp.ones((128,), jnp.float32)
    z1 = jnp.zeros((112, D1), jnp.float32)
    z2 = jnp.zeros((32, D2), jnp.float32)

    H = _sc_degree(dst_p, zdeg, ones16)
    dinv, u0 = _tc_prep(H, xp)
    S0 = _sc_segsum_1(u0, src_p, dst_p, z1)[:NP]
    u1 = _tc_layer1(S0, u0, dinv, W1p, b1p)
    S1 = _sc_segsum_2(u1, src_p, dst_p, z2)[:NP]
    out = _tc_final(S1, u1, dinv, W2p, b2, fc1_W, fc1_b, fc2_W, fc2_b)
    return out.reshape((1,))


# speedup vs baseline: 9.1898x; 1.2076x over previous
import functools

import jax
import jax.numpy as jnp
from jax import lax
from jax.experimental import pallas as pl
from jax.experimental.pallas import tpu as pltpu
from jax.experimental.pallas import tpu_sc as plsc

N = 50000
E = 800000
NP = 50176
EP = 819200
CH = 2048
NCHUNK = 25
DCH = 1280
NDCH = 20
D1 = 256
D2 = 896

_mesh = plsc.VectorSubcoreMesh(core_axis_name="c", subcore_axis_name="s")
_sc_params = pltpu.CompilerParams(needs_layout_passes=False)


def _sc_degree(dst_p, zdeg, ones16):
    FST = NP // 8

    @functools.partial(
        pl.kernel,
        out_type=jax.ShapeDtypeStruct((2 * NP,), jnp.float32),
        mesh=_mesh,
        compiler_params=_sc_params,
        scratch_types=[
            pltpu.VMEM((DCH,), jnp.int32),
            pltpu.VMEM((128,), jnp.float32),
            pltpu.VMEM((NP // 8,), jnp.float32),
            pltpu.VMEM_SHARED((NP,), jnp.float32),
        ],
    )
    def deg_kernel(dst_hbm, zdeg_hbm, ones_hbm, out_hbm, dbuf, ones_v,
                   hbounce, hist):
        core = lax.axis_index("c")
        sub = lax.axis_index("s")
        wid = sub * 2 + core

        @pl.when(sub < 8)
        def _():
            pltpu.sync_copy(zdeg_hbm, hbounce)
            pltpu.sync_copy(hbounce, hist.at[pl.ds(sub * FST, FST)])

        pltpu.sync_copy(ones_hbm, ones_v)
        plsc.subcore_barrier()
        base = wid * (EP // 32)

        def chunk(c, carry):
            pltpu.sync_copy(dst_hbm.at[pl.ds(base + c * DCH, DCH)], dbuf)

            def grp(g, carry2):
                idx = dbuf.at[pl.ds(g * 128, 128)]
                pltpu.sync_copy(ones_v, hist.at[idx], add=True)
                return carry2

            return lax.fori_loop(0, DCH // 128, grp, carry)

        lax.fori_loop(0, NDCH, chunk, jnp.int32(0))
        plsc.subcore_barrier()

        @pl.when(sub < 8)
        def _():
            pltpu.sync_copy(hist.at[pl.ds(sub * FST, FST)], hbounce)
            pltpu.sync_copy(
                hbounce, out_hbm.at[pl.ds(core * NP + sub * FST, FST)])

    return deg_kernel(dst_p, zdeg, ones16)


def _make_sc_segsum(D, FT, ZB, NZ):
    OWN = NP // 32

    @functools.partial(
        pl.kernel,
        out_type=jax.ShapeDtypeStruct((NP + 32, D), jnp.float32),
        mesh=_mesh,
        compiler_params=_sc_params,
        scratch_types=[
            pltpu.VMEM((CH,), jnp.int32),
            pltpu.VMEM((CH,), jnp.int32),
            pltpu.VMEM((CH,), jnp.int32),
            pltpu.VMEM((CH,), jnp.int32),
            pltpu.VMEM((CH + FT,), jnp.int32),
            pltpu.VMEM((CH + FT,), jnp.int32),
            pltpu.VMEM((FT, D), jnp.float32),
            pltpu.VMEM((ZB, D), jnp.float32),
            pltpu.SemaphoreType.DMA,
            pltpu.SemaphoreType.DMA,
        ],
    )
    def seg_kernel(u_hbm, src_hbm, dst_hbm, zblk_hbm, out_hbm,
                   ebuf_s0, ebuf_d0, ebuf_s1, ebuf_d1, slist, dlist,
                   stage, zbuf, sem0, sem1):
        core = lax.axis_index("c")
        sub = lax.axis_index("s")
        wid = sub * 2 + core
        lo = wid * OWN
        garbage = NP + wid
        pltpu.sync_copy(zblk_hbm, zbuf)
        for z in range(NZ):
            pltpu.sync_copy(zbuf, out_hbm.at[pl.ds(lo + z * ZB, ZB)])

        pltpu.async_copy(src_hbm.at[pl.ds(0, CH)], ebuf_s0, sem0)
        pltpu.async_copy(dst_hbm.at[pl.ds(0, CH)], ebuf_d0, sem0)
        NC = EP // CH

        def scan_flush(ebuf_s, ebuf_d, ptr_in):
            def per_group(g, ptr):
                dv = ebuf_d[pl.ds(g * 16, 16)]
                sv = ebuf_s[pl.ds(g * 16, 16)]
                m = plsc.bitcast(dv - lo, jnp.uint32) < jnp.uint32(OWN)
                mi = m.astype(jnp.int32)
                cum = plsc.cumsum(mi)
                pos = ptr + cum - 1
                plsc.store_scatter(slist, [pos], sv, mask=m)
                plsc.store_scatter(dlist, [pos], dv, mask=m)
                return ptr + cum[15]

            ptr = lax.fori_loop(0, CH // 16, per_group, ptr_in)
            k = ptr // FT

            def flush(j, carry):
                sref = slist.at[pl.ds(j * FT, FT)]
                dref = dlist.at[pl.ds(j * FT, FT)]
                pltpu.sync_copy(u_hbm.at[sref], stage)
                pltpu.sync_copy(stage, out_hbm.at[dref], add=True)
                return carry

            lax.fori_loop(0, k, flush, jnp.int32(0))
            for t in range(FT // 16):
                slist[pl.ds(t * 16, 16)] = slist[pl.ds(k * FT + t * 16, 16)]
                dlist[pl.ds(t * 16, 16)] = dlist[pl.ds(k * FT + t * 16, 16)]
            return ptr - k * FT

        def per_chunk(c, ptr_in):
            def side(bs, bd, semc, bsn, bdn, semn):
                pltpu.make_async_copy(
                    src_hbm.at[pl.ds(0, CH)], bs, semc).wait()
                pltpu.make_async_copy(
                    dst_hbm.at[pl.ds(0, CH)], bd, semc).wait()

                @pl.when(c + 1 < NC)
                def _():
                    pltpu.async_copy(
                        src_hbm.at[pl.ds((c + 1) * CH, CH)], bsn, semn)
                    pltpu.async_copy(
                        dst_hbm.at[pl.ds((c + 1) * CH, CH)], bdn, semn)

                return scan_flush(bs, bd, ptr_in)

            return lax.cond(
                c % 2 == 0,
                lambda: side(ebuf_s0, ebuf_d0, sem0, ebuf_s1, ebuf_d1, sem1),
                lambda: side(ebuf_s1, ebuf_d1, sem1, ebuf_s0, ebuf_d0, sem0),
            )

        ptrf = lax.fori_loop(0, NC, per_chunk, jnp.int32(0))
        lane = lax.iota(jnp.int32, 16)
        for t in range(FT // 16):
            mt = (lane + t * 16) < ptrf
            slist[pl.ds(t * 16, 16)] = jnp.where(
                mt, slist[pl.ds(t * 16, 16)], 0)
            dlist[pl.ds(t * 16, 16)] = jnp.where(
                mt, dlist[pl.ds(t * 16, 16)], garbage)
        pltpu.sync_copy(u_hbm.at[slist.at[pl.ds(0, FT)]], stage)
        pltpu.sync_copy(stage, out_hbm.at[dlist.at[pl.ds(0, FT)]], add=True)

    return seg_kernel


_sc_segsum_1 = _make_sc_segsum(D1, 64, 112, 14)
_sc_segsum_2 = _make_sc_segsum(D2, 64, 32, 49)


def _tc_prep_body(dega_ref, degb_ref, x_ref, dinv_ref, u0_ref):
    deg = dega_ref[...] + degb_ref[...] + 1.0
    dv = lax.rsqrt(deg)
    dinv_ref[...] = dv
    u0_ref[...] = x_ref[...] * dv


def _tc_prep(H, xp):
    R, G = 3136, 16
    degc = H.reshape(2 * NP, 1)
    return pl.pallas_call(
        _tc_prep_body,
        grid=(G,),
        in_specs=[
            pl.BlockSpec((R, 1), lambda i: (i, 0)),
            pl.BlockSpec((R, 1), lambda i: (i + 16, 0)),
            pl.BlockSpec((R, D1), lambda i: (i, 0)),
        ],
        out_specs=[
            pl.BlockSpec((R, 1), lambda i: (i, 0)),
            pl.BlockSpec((R, D1), lambda i: (i, 0)),
        ],
        out_shape=[
            jax.ShapeDtypeStruct((NP, 1), jnp.float32),
            jax.ShapeDtypeStruct((NP, D1), jnp.float32),
        ],
    )(degc, degc, xp)


def _tc_layer1_body(s0_ref, u0_ref, dinv_ref, w1_ref, b1_ref, u1_ref):
    agg = dinv_ref[...] * (s0_ref[...] + u0_ref[...])
    z = jnp.dot(agg, w1_ref[...], preferred_element_type=jnp.float32,
                precision=lax.Precision.HIGHEST)
    h = jnp.maximum(z + b1_ref[...], 0.0)
    u1_ref[...] = dinv_ref[...] * h


def _tc_layer1(S0, u0, dinv, W1p, b1p):
    R, G = 1568, 32
    return pl.pallas_call(
        _tc_layer1_body,
        grid=(G,),
        in_specs=[
            pl.BlockSpec((R, D1), lambda i: (i, 0)),
            pl.BlockSpec((R, D1), lambda i: (i, 0)),
            pl.BlockSpec((R, 1), lambda i: (i, 0)),
            pl.BlockSpec((D1, D2), lambda i: (0, 0)),
            pl.BlockSpec((1, D2), lambda i: (0, 0)),
        ],
        out_specs=pl.BlockSpec((R, D2), lambda i: (i, 0)),
        out_shape=jax.ShapeDtypeStruct((NP, D2), jnp.float32),
    )(S0, u0, dinv, W1p, b1p)


def _make_tc_final_body(R, G):
    def body(s1_ref, u1_ref, dinv_ref, w2_ref, b2_ref,
             f1w_ref, f1b_ref, f2w_ref, f2b_ref, out_ref, acc_ref):
        i = pl.program_id(0)

        @pl.when(i == 0)
        def _():
            acc_ref[...] = jnp.zeros_like(acc_ref)

        agg = dinv_ref[...] * (s1_ref[...] + u1_ref[...])
        z = jnp.dot(agg, w2_ref[...], preferred_element_type=jnp.float32,
                    precision=lax.Precision.HIGHEST)
        h = jnp.maximum(z + b2_ref[...], 0.0)
        rid = i * R + lax.broadcasted_iota(jnp.int32, (R, 1), 0)
        h = jnp.where(rid < N, h, 0.0)
        acc_ref[...] += jnp.sum(h, axis=0, keepdims=True)

        @pl.when(i == G - 1)
        def _():
            m = acc_ref[...] / float(N)
            t = jnp.dot(m, f1w_ref[...], preferred_element_type=jnp.float32,
                        precision=lax.Precision.HIGHEST)
            t = t + f1b_ref[...]
            t = jnp.where(t > 0, t, 0.2 * t)
            o = jnp.dot(t, f2w_ref[...], preferred_element_type=jnp.float32,
                        precision=lax.Precision.HIGHEST)
            o = o + f2b_ref[...]
            out_ref[...] = 1.0 / (1.0 + jnp.exp(-o))

    return body


def _tc_final(S1, u1, dinv, W2p, b2, fc1_W, fc1_b, fc2_W, fc2_b):
    R, G = 1568, 32
    return pl.pallas_call(
        _make_tc_final_body(R, G),
        grid=(G,),
        in_specs=[
            pl.BlockSpec((R, D2), lambda i: (i, 0)),
            pl.BlockSpec((R, D2), lambda i: (i, 0)),
            pl.BlockSpec((R, 1), lambda i: (i, 0)),
            pl.BlockSpec((D2, 1024), lambda i: (0, 0)),
            pl.BlockSpec((1, 1024), lambda i: (0, 0)),
            pl.BlockSpec((1024, 512), lambda i: (0, 0)),
            pl.BlockSpec((1, 512), lambda i: (0, 0)),
            pl.BlockSpec((512, 1), lambda i: (0, 0)),
            pl.BlockSpec((1, 1), lambda i: (0, 0)),
        ],
        out_specs=pl.BlockSpec((1, 1), lambda i: (0, 0)),
        out_shape=jax.ShapeDtypeStruct((1, 1), jnp.float32),
        scratch_shapes=[pltpu.VMEM((1, 1024), jnp.float32)],
    )(S1, u1, dinv, W2p, b2.reshape(1, 1024), fc1_W,
      fc1_b.reshape(1, 512), fc2_W, fc2_b.reshape(1, 1))


def kernel(x, edge_index, W1, b1, W2, b2, fc1_W, fc1_b, fc2_W, fc2_b):
    xp = jnp.pad(x, ((0, NP - N), (0, D1 - 78)))
    src_p = jnp.pad(edge_index[0], (0, EP - E), constant_values=0)
    dst_p = jnp.pad(edge_index[1], (0, EP - E), constant_values=NP)
    W1p = jnp.pad(W1, ((0, D1 - 78), (0, D2 - 780)))
    b1p = jnp.pad(b1, (0, D2 - 780)).reshape(1, D2)
    W2p = jnp.pad(W2, ((0, D2 - 780), (0, 0)))

    zdeg = jnp.zeros((NP // 8,), jnp.float32)
    ones16 = jnp.ones((128,), jnp.float32)
    z1 = jnp.zeros((112, D1), jnp.float32)
    z2 = jnp.zeros((32, D2), jnp.float32)

    H = _sc_degree(dst_p, zdeg, ones16)
    dinv, u0 = _tc_prep(H, xp)
    S0 = _sc_segsum_1(u0, src_p, dst_p, z1)[:NP]
    u1 = _tc_layer1(S0, u0, dinv, W1p, b1p)
    S1 = _sc_segsum_2(u1, src_p, dst_p, z2)[:NP]
    out = _tc_final(S1, u1, dinv, W2p, b2, fc1_W, fc1_b, fc2_W, fc2_b)
    return out.reshape((1,))


# speedup vs baseline: 10.1430x; 1.1037x over previous
import functools

import jax
import jax.numpy as jnp
from jax import lax
from jax.experimental import pallas as pl
from jax.experimental.pallas import tpu as pltpu
from jax.experimental.pallas import tpu_sc as plsc

N = 50000
E = 800000
NP = 50176
EP = 819200
CH = 2048
NCHUNK = 25
DCH = 1280
NDCH = 20
D1 = 256
D2 = 896

_mesh = plsc.VectorSubcoreMesh(core_axis_name="c", subcore_axis_name="s")
_sc_params = pltpu.CompilerParams(needs_layout_passes=False)


def _sc_degree(dst_p, zdeg, ones16):
    FST = NP // 8

    @functools.partial(
        pl.kernel,
        out_type=jax.ShapeDtypeStruct((2 * NP,), jnp.float32),
        mesh=_mesh,
        compiler_params=_sc_params,
        scratch_types=[
            pltpu.VMEM((DCH,), jnp.int32),
            pltpu.VMEM((128,), jnp.float32),
            pltpu.VMEM((NP // 8,), jnp.float32),
            pltpu.VMEM_SHARED((NP,), jnp.float32),
        ],
    )
    def deg_kernel(dst_hbm, zdeg_hbm, ones_hbm, out_hbm, dbuf, ones_v,
                   hbounce, hist):
        core = lax.axis_index("c")
        sub = lax.axis_index("s")
        wid = sub * 2 + core

        @pl.when(sub < 8)
        def _():
            pltpu.sync_copy(zdeg_hbm, hbounce)
            pltpu.sync_copy(hbounce, hist.at[pl.ds(sub * FST, FST)])

        pltpu.sync_copy(ones_hbm, ones_v)
        plsc.subcore_barrier()
        base = wid * (EP // 32)

        def chunk(c, carry):
            pltpu.sync_copy(dst_hbm.at[pl.ds(base + c * DCH, DCH)], dbuf)

            def grp(g, carry2):
                idx = dbuf.at[pl.ds(g * 128, 128)]
                pltpu.sync_copy(ones_v, hist.at[idx], add=True)
                return carry2

            return lax.fori_loop(0, DCH // 128, grp, carry)

        lax.fori_loop(0, NDCH, chunk, jnp.int32(0))
        plsc.subcore_barrier()

        @pl.when(sub < 8)
        def _():
            pltpu.sync_copy(hist.at[pl.ds(sub * FST, FST)], hbounce)
            pltpu.sync_copy(
                hbounce, out_hbm.at[pl.ds(core * NP + sub * FST, FST)])

    return deg_kernel(dst_p, zdeg, ones16)


def _make_sc_segsum(D, FT, ZB, NZ):
    OWN = NP // 32

    @functools.partial(
        pl.kernel,
        out_type=jax.ShapeDtypeStruct((NP + 32, D), jnp.float32),
        mesh=_mesh,
        compiler_params=_sc_params,
        scratch_types=[
            pltpu.VMEM((CH,), jnp.int32),
            pltpu.VMEM((CH,), jnp.int32),
            pltpu.VMEM((CH,), jnp.int32),
            pltpu.VMEM((CH,), jnp.int32),
            pltpu.VMEM((CH + FT,), jnp.int32),
            pltpu.VMEM((CH + FT,), jnp.int32),
            pltpu.VMEM((FT, D), jnp.float32),
            pltpu.VMEM((FT, D), jnp.float32),
            pltpu.VMEM((ZB, D), jnp.float32),
            pltpu.SemaphoreType.DMA,
            pltpu.SemaphoreType.DMA,
            pltpu.SemaphoreType.DMA,
        ],
    )
    def seg_kernel(u_hbm, src_hbm, dst_hbm, zblk_hbm, out_hbm,
                   ebuf_s0, ebuf_d0, ebuf_s1, ebuf_d1, slist, dlist,
                   stage0, stage1, zbuf, sem0, sem1, sem_sc):
        core = lax.axis_index("c")
        sub = lax.axis_index("s")
        wid = sub * 2 + core
        lo = wid * OWN
        garbage = NP + wid
        pltpu.sync_copy(zblk_hbm, zbuf)
        for z in range(NZ):
            pltpu.sync_copy(zbuf, out_hbm.at[pl.ds(lo + z * ZB, ZB)])

        pltpu.async_copy(src_hbm.at[pl.ds(0, CH)], ebuf_s0, sem0)
        pltpu.async_copy(dst_hbm.at[pl.ds(0, CH)], ebuf_d0, sem0)
        NC = EP // CH

        def do_flush(sref, dref, fc):
            def side(stg, oth):
                pltpu.sync_copy(u_hbm.at[sref], stg)

                @pl.when(fc >= 1)
                def _():
                    pltpu.make_async_copy(oth, out_hbm.at[dref], sem_sc).wait()

                pltpu.async_copy(stg, out_hbm.at[dref], sem_sc, add=True)

            lax.cond(fc % 2 == 0,
                     lambda: side(stage0, stage1),
                     lambda: side(stage1, stage0))
            return fc + 1

        def scan_flush(ebuf_s, ebuf_d, ptr_in, fc_in):
            def per_group(g, ptr):
                dv = ebuf_d[pl.ds(g * 16, 16)]
                sv = ebuf_s[pl.ds(g * 16, 16)]
                m = plsc.bitcast(dv - lo, jnp.uint32) < jnp.uint32(OWN)
                mi = m.astype(jnp.int32)
                cum = plsc.cumsum(mi)
                pos = ptr + cum - 1
                plsc.store_scatter(slist, [pos], sv, mask=m)
                plsc.store_scatter(dlist, [pos], dv, mask=m)
                return ptr + cum[15]

            ptr = lax.fori_loop(0, CH // 16, per_group, ptr_in)
            k = ptr // FT

            def flush(j, fc):
                sref = slist.at[pl.ds(j * FT, FT)]
                dref = dlist.at[pl.ds(j * FT, FT)]
                return do_flush(sref, dref, fc)

            fc_out = lax.fori_loop(0, k, flush, fc_in)
            for t in range(FT // 16):
                slist[pl.ds(t * 16, 16)] = slist[pl.ds(k * FT + t * 16, 16)]
                dlist[pl.ds(t * 16, 16)] = dlist[pl.ds(k * FT + t * 16, 16)]
            return ptr - k * FT, fc_out

        def per_chunk(c, carry_in):
            def side(bs, bd, semc, bsn, bdn, semn):
                pltpu.make_async_copy(
                    src_hbm.at[pl.ds(0, CH)], bs, semc).wait()
                pltpu.make_async_copy(
                    dst_hbm.at[pl.ds(0, CH)], bd, semc).wait()

                @pl.when(c + 1 < NC)
                def _():
                    pltpu.async_copy(
                        src_hbm.at[pl.ds((c + 1) * CH, CH)], bsn, semn)
                    pltpu.async_copy(
                        dst_hbm.at[pl.ds((c + 1) * CH, CH)], bdn, semn)

                return scan_flush(bs, bd, ptr_in, fc_in)

            ptr_in, fc_in = carry_in
            return lax.cond(
                c % 2 == 0,
                lambda: side(ebuf_s0, ebuf_d0, sem0, ebuf_s1, ebuf_d1, sem1),
                lambda: side(ebuf_s1, ebuf_d1, sem1, ebuf_s0, ebuf_d0, sem0),
            )

        ptrf, fcf = lax.fori_loop(
            0, NC, per_chunk, (jnp.int32(0), jnp.int32(0)))
        lane = lax.iota(jnp.int32, 16)
        for t in range(FT // 16):
            mt = (lane + t * 16) < ptrf
            slist[pl.ds(t * 16, 16)] = jnp.where(
                mt, slist[pl.ds(t * 16, 16)], 0)
            dlist[pl.ds(t * 16, 16)] = jnp.where(
                mt, dlist[pl.ds(t * 16, 16)], garbage)
        fcf = do_flush(slist.at[pl.ds(0, FT)], dlist.at[pl.ds(0, FT)], fcf)

        def drain(stg):
            pltpu.make_async_copy(
                stg, out_hbm.at[dlist.at[pl.ds(0, FT)]], sem_sc).wait()

        lax.cond((fcf - 1) % 2 == 0,
                 lambda: drain(stage0), lambda: drain(stage1))

    return seg_kernel


_sc_segsum_1 = _make_sc_segsum(D1, 64, 112, 14)
_sc_segsum_2 = _make_sc_segsum(D2, 32, 32, 49)


def _tc_prep_body(dega_ref, degb_ref, x_ref, dinv_ref, u0_ref):
    deg = dega_ref[...] + degb_ref[...] + 1.0
    dv = lax.rsqrt(deg)
    dinv_ref[...] = dv
    u0_ref[...] = x_ref[...] * dv


def _tc_prep(H, xp):
    R, G = 3136, 16
    degc = H.reshape(2 * NP, 1)
    return pl.pallas_call(
        _tc_prep_body,
        grid=(G,),
        in_specs=[
            pl.BlockSpec((R, 1), lambda i: (i, 0)),
            pl.BlockSpec((R, 1), lambda i: (i + 16, 0)),
            pl.BlockSpec((R, D1), lambda i: (i, 0)),
        ],
        out_specs=[
            pl.BlockSpec((R, 1), lambda i: (i, 0)),
            pl.BlockSpec((R, D1), lambda i: (i, 0)),
        ],
        out_shape=[
            jax.ShapeDtypeStruct((NP, 1), jnp.float32),
            jax.ShapeDtypeStruct((NP, D1), jnp.float32),
        ],
    )(degc, degc, xp)


def _tc_layer1_body(s0_ref, u0_ref, dinv_ref, w1_ref, b1_ref, u1_ref):
    agg = dinv_ref[...] * (s0_ref[...] + u0_ref[...])
    z = jnp.dot(agg, w1_ref[...], preferred_element_type=jnp.float32,
                precision=lax.Precision.HIGHEST)
    h = jnp.maximum(z + b1_ref[...], 0.0)
    u1_ref[...] = dinv_ref[...] * h


def _tc_layer1(S0, u0, dinv, W1p, b1p):
    R, G = 1568, 32
    return pl.pallas_call(
        _tc_layer1_body,
        grid=(G,),
        in_specs=[
            pl.BlockSpec((R, D1), lambda i: (i, 0)),
            pl.BlockSpec((R, D1), lambda i: (i, 0)),
            pl.BlockSpec((R, 1), lambda i: (i, 0)),
            pl.BlockSpec((D1, D2), lambda i: (0, 0)),
            pl.BlockSpec((1, D2), lambda i: (0, 0)),
        ],
        out_specs=pl.BlockSpec((R, D2), lambda i: (i, 0)),
        out_shape=jax.ShapeDtypeStruct((NP, D2), jnp.float32),
    )(S0, u0, dinv, W1p, b1p)


def _make_tc_final_body(R, G):
    def body(s1_ref, u1_ref, dinv_ref, w2_ref, b2_ref,
             f1w_ref, f1b_ref, f2w_ref, f2b_ref, out_ref, acc_ref):
        i = pl.program_id(0)

        @pl.when(i == 0)
        def _():
            acc_ref[...] = jnp.zeros_like(acc_ref)

        agg = dinv_ref[...] * (s1_ref[...] + u1_ref[...])
        z = jnp.dot(agg, w2_ref[...], preferred_element_type=jnp.float32,
                    precision=lax.Precision.HIGHEST)
        h = jnp.maximum(z + b2_ref[...], 0.0)
        rid = i * R + lax.broadcasted_iota(jnp.int32, (R, 1), 0)
        h = jnp.where(rid < N, h, 0.0)
        acc_ref[...] += jnp.sum(h, axis=0, keepdims=True)

        @pl.when(i == G - 1)
        def _():
            m = acc_ref[...] / float(N)
            t = jnp.dot(m, f1w_ref[...], preferred_element_type=jnp.float32,
                        precision=lax.Precision.HIGHEST)
            t = t + f1b_ref[...]
            t = jnp.where(t > 0, t, 0.2 * t)
            o = jnp.dot(t, f2w_ref[...], preferred_element_type=jnp.float32,
                        precision=lax.Precision.HIGHEST)
            o = o + f2b_ref[...]
            out_ref[...] = 1.0 / (1.0 + jnp.exp(-o))

    return body


def _tc_final(S1, u1, dinv, W2p, b2, fc1_W, fc1_b, fc2_W, fc2_b):
    R, G = 1568, 32
    return pl.pallas_call(
        _make_tc_final_body(R, G),
        grid=(G,),
        in_specs=[
            pl.BlockSpec((R, D2), lambda i: (i, 0)),
            pl.BlockSpec((R, D2), lambda i: (i, 0)),
            pl.BlockSpec((R, 1), lambda i: (i, 0)),
            pl.BlockSpec((D2, 1024), lambda i: (0, 0)),
            pl.BlockSpec((1, 1024), lambda i: (0, 0)),
            pl.BlockSpec((1024, 512), lambda i: (0, 0)),
            pl.BlockSpec((1, 512), lambda i: (0, 0)),
            pl.BlockSpec((512, 1), lambda i: (0, 0)),
            pl.BlockSpec((1, 1), lambda i: (0, 0)),
        ],
        out_specs=pl.BlockSpec((1, 1), lambda i: (0, 0)),
        out_shape=jax.ShapeDtypeStruct((1, 1), jnp.float32),
        scratch_shapes=[pltpu.VMEM((1, 1024), jnp.float32)],
    )(S1, u1, dinv, W2p, b2.reshape(1, 1024), fc1_W,
      fc1_b.reshape(1, 512), fc2_W, fc2_b.reshape(1, 1))


def kernel(x, edge_index, W1, b1, W2, b2, fc1_W, fc1_b, fc2_W, fc2_b):
    xp = jnp.pad(x, ((0, NP - N), (0, D1 - 78)))
    src_p = jnp.pad(edge_index[0], (0, EP - E), constant_values=0)
    dst_p = jnp.pad(edge_index[1], (0, EP - E), constant_values=NP)
    W1p = jnp.pad(W1, ((0, D1 - 78), (0, D2 - 780)))
    b1p = jnp.pad(b1, (0, D2 - 780)).reshape(1, D2)
    W2p = jnp.pad(W2, ((0, D2 - 780), (0, 0)))

    zdeg = jnp.zeros((NP // 8,), jnp.float32)
    ones16 = jnp.ones((128,), jnp.float32)
    z1 = jnp.zeros((112, D1), jnp.float32)
    z2 = jnp.zeros((32, D2), jnp.float32)

    H = _sc_degree(dst_p, zdeg, ones16)
    dinv, u0 = _tc_prep(H, xp)
    S0 = _sc_segsum_1(u0, src_p, dst_p, z1)[:NP]
    u1 = _tc_layer1(S0, u0, dinv, W1p, b1p)
    S1 = _sc_segsum_2(u1, src_p, dst_p, z2)[:NP]
    out = _tc_final(S1, u1, dinv, W2p, b2, fc1_W, fc1_b, fc2_W, fc2_b)
    return out.reshape((1,))
